# baseline (speedup 1.0000x reference)
import functools

import jax
import jax.numpy as jnp
from jax import lax
from jax.experimental import pallas as pl
from jax.experimental.pallas import tpu as pltpu
from jax.experimental.pallas import tpu_sc as plsc

B = 16384
NUM_DENSE = 13
NUM_CAT = 26
NUM_SEQ = 2
VOCAB = 100000
EMB = 16
SEQ_LEN = 50
HID1 = 96
HID2 = 32
CAT_COLS = NUM_CAT * EMB

NW = 32
ROWS_W = B // NW

KT_CHUNK = 2048
KT_FULL = 48
KT_TAILV = 1664
RPF = 12504
VPAD = RPF * 8

CAT_SUB = 128
CAT_CHUNK = CAT_SUB * NUM_CAT
SEQ_SUB = 64
SEQ_CHUNK = SEQ_SUB * SEQ_LEN

BLK = 512
NBLK = B // BLK


@functools.cache
def _make_kt():
    mesh = plsc.VectorSubcoreMesh(core_axis_name="c", subcore_axis_name="s")
    return functools.partial(
        pl.kernel,
        out_type=(
            jax.ShapeDtypeStruct((NUM_CAT, RPF, 128), jnp.float32),
            jax.ShapeDtypeStruct((NUM_SEQ, RPF, 128), jnp.float32),
        ),
        mesh=mesh,
        scratch_types=[
            pltpu.VMEM((EMB, KT_CHUNK), jnp.float32),
            pltpu.VMEM((KT_CHUNK // 8, 128), jnp.float32),
            pltpu.VMEM((8, 128), jnp.float32),
            pltpu.SemaphoreType.DMA,
        ],
        compiler_params=pltpu.CompilerParams(
            use_tc_tiling_on_sc=True, needs_layout_passes=False),
    )(_kt_body)


def _kt_field(tab_hbm, app_hbm, out_hbm, f, vin, vout, vtail):
    iota16 = lax.iota(jnp.int32, 16)

    def transpose_chunk(nr):
        def row(r, _):
            base = lax.broadcast_in_dim(r * 8, (16,), ())
            for j in range(8):
                col = plsc.load_gather(vin, [iota16, base + j])
                vout[r, pl.ds(j * EMB, EMB)] = col
            return 0
        lax.fori_loop(0, nr, row, 0)

    def chunk(c, _):
        v0 = c * KT_CHUNK
        pltpu.sync_copy(tab_hbm.at[f, pl.ds(0, 8), pl.ds(v0, KT_CHUNK)],
                        vin.at[pl.ds(0, 8), :])
        pltpu.sync_copy(tab_hbm.at[f, pl.ds(8, 8), pl.ds(v0, KT_CHUNK)],
                        vin.at[pl.ds(8, 8), :])
        transpose_chunk(KT_CHUNK // 8)
        pltpu.sync_copy(vout,
                        out_hbm.at[f, pl.ds(c * (KT_CHUNK // 8),
                                            KT_CHUNK // 8), :])
        return 0

    lax.fori_loop(0, KT_FULL, chunk, 0)
    v0 = KT_FULL * KT_CHUNK
    pltpu.sync_copy(tab_hbm.at[f, pl.ds(0, 8), pl.ds(v0, KT_TAILV)],
                    vin.at[pl.ds(0, 8), pl.ds(0, KT_TAILV)])
    pltpu.sync_copy(tab_hbm.at[f, pl.ds(8, 8), pl.ds(v0, KT_TAILV)],
                    vin.at[pl.ds(8, 8), pl.ds(0, KT_TAILV)])
    transpose_chunk(KT_TAILV // 8)
    pltpu.sync_copy(vout.at[pl.ds(0, KT_TAILV // 8), :],
                    out_hbm.at[f, pl.ds(v0 // 8, KT_TAILV // 8), :])
    pltpu.sync_copy(app_hbm.at[f], vtail)
    pltpu.sync_copy(vtail, out_hbm.at[f, pl.ds(12496, 8), :])


def _kt_body(cat_tabT_hbm, seq_tabT_hbm, app_cat_hbm, app_seq_hbm,
             cat_out_hbm, seq_out_hbm, vin, vout, vtail, sem):
    wid = lax.axis_index("s") * 2 + lax.axis_index("c")

    @pl.when(wid < NUM_CAT)
    def _():
        _kt_field(cat_tabT_hbm, app_cat_hbm, cat_out_hbm, wid,
                  vin, vout, vtail)

    @pl.when(jnp.logical_and(wid >= NUM_CAT, wid < NUM_CAT + NUM_SEQ))
    def _():
        _kt_field(seq_tabT_hbm, app_seq_hbm, seq_out_hbm, wid - NUM_CAT,
                  vin, vout, vtail)


@functools.cache
def _make_kg():
    mesh = plsc.VectorSubcoreMesh(core_axis_name="c", subcore_axis_name="s")
    return functools.partial(
        pl.kernel,
        out_type=(
            jax.ShapeDtypeStruct((B * NUM_CAT, EMB), jnp.float32),
            jax.ShapeDtypeStruct((B, NUM_SEQ * EMB), jnp.float32),
        ),
        mesh=mesh,
        scratch_types=[
            pltpu.VMEM((CAT_CHUNK,), jnp.int32),
            pltpu.VMEM((CAT_CHUNK, EMB), jnp.float32),
            pltpu.VMEM((SEQ_CHUNK,), jnp.int32),
            pltpu.VMEM((SEQ_CHUNK, EMB), jnp.float32),
            pltpu.VMEM((SEQ_SUB, NUM_SEQ * EMB), jnp.float32),
            pltpu.SemaphoreType.DMA,
        ],
        compiler_params=pltpu.CompilerParams(use_tc_tiling_on_sc=False),
    )(_kg_body)


def _kg_body(cat_idx_hbm, seq_idx_hbm, cat_tab_hbm, seq_tab_hbm,
             cat_out_hbm, seq_out_hbm,
             cidx_v, crows_v, sidx_v, srows_v, pool_v, sem):
    wid = lax.axis_index("s") * 2 + lax.axis_index("c")
    base = wid * ROWS_W

    for k in range(ROWS_W // CAT_SUB):
        off = (base + k * CAT_SUB) * NUM_CAT
        pltpu.sync_copy(cat_idx_hbm.at[pl.ds(off, CAT_CHUNK)], cidx_v)
        pltpu.async_copy(cat_tab_hbm.at[cidx_v], crows_v, sem).wait()
        pltpu.sync_copy(crows_v, cat_out_hbm.at[pl.ds(off, CAT_CHUNK), :])

    for j in range(ROWS_W // SEQ_SUB):
        sub = base + j * SEQ_SUB
        for s in range(NUM_SEQ):
            pltpu.sync_copy(
                seq_idx_hbm.at[s, pl.ds(sub * SEQ_LEN, SEQ_CHUNK)], sidx_v)
            pltpu.async_copy(seq_tab_hbm.at[s].at[sidx_v], srows_v,
                             sem).wait()

            def _pool_row(i, _):
                a0 = srows_v[i * SEQ_LEN, :]
                a1 = srows_v[i * SEQ_LEN + 1, :]
                a2 = srows_v[i * SEQ_LEN + 2, :]
                a3 = srows_v[i * SEQ_LEN + 3, :]
                for l in range(4, SEQ_LEN, 4):
                    a0 = a0 + srows_v[i * SEQ_LEN + l, :]
                    a1 = a1 + srows_v[i * SEQ_LEN + l + 1, :]
                    a2 = a2 + srows_v[i * SEQ_LEN + l + 2, :]
                    a3 = a3 + srows_v[i * SEQ_LEN + l + 3, :]
                pool_v[i, pl.ds(s * EMB, EMB)] = (a0 + a1) + (a2 + a3)
                return 0

            lax.fori_loop(0, SEQ_SUB, _pool_row, 0)
        pltpu.sync_copy(pool_v, seq_out_hbm.at[pl.ds(sub, SEQ_SUB), :])


def _mlp1_body(dense_ref, cat_ref, seq_ref, w1d_ref, w1c_ref, w1s_ref, b1_ref,
               wfw_ref, wfx_ref, bf_ref,
               x1_ref, plog_ref, s1_ref, q1_ref):
    i = pl.program_id(0)
    dense = dense_ref[...]
    cat = cat_ref[...]
    seq = seq_ref[...]
    x1 = (jnp.dot(dense, w1d_ref[...], preferred_element_type=jnp.float32)
          + jnp.dot(cat, w1c_ref[...], preferred_element_type=jnp.float32)
          + jnp.dot(seq, w1s_ref[...], preferred_element_type=jnp.float32)
          + b1_ref[...])
    x1_ref[...] = x1

    cross = jnp.concatenate(
        [cat[:, 0:16] * cat[:, 16:32], cat[:, 32:48] * cat[:, 48:64],
         cat[:, 64:80] * cat[:, 80:96], cat[:, 96:112] * cat[:, 112:128]],
        axis=1)
    plog = (jnp.sum(dense * wfw_ref[:, 0:NUM_DENSE], axis=1, keepdims=True)
            + jnp.sum(cat * wfw_ref[:, NUM_DENSE:], axis=1, keepdims=True)
            + jnp.sum(cross * wfx_ref[...], axis=1, keepdims=True)
            + bf_ref[...])
    plog_ref[...] = plog

    @pl.when(i == 0)
    def _():
        s1_ref[...] = jnp.zeros_like(s1_ref)
        q1_ref[...] = jnp.zeros_like(q1_ref)
    s1_ref[...] += jnp.sum(x1, axis=0, keepdims=True)
    q1_ref[...] += jnp.sum(x1 * x1, axis=0, keepdims=True)


def _mlp2_body(x1_ref, s1_ref, q1_ref, w2_ref, b2_ref,
               x2_ref, s2_ref, q2_ref):
    i = pl.program_id(0)
    m = s1_ref[...] / B
    v = q1_ref[...] / B - m * m
    inv = lax.rsqrt(v + 1e-5)
    h1 = jnp.maximum((x1_ref[...] - m) * inv, 0.0)
    x2 = jnp.dot(h1, w2_ref[...], preferred_element_type=jnp.float32) + b2_ref[...]
    x2_ref[...] = x2

    @pl.when(i == 0)
    def _():
        s2_ref[...] = jnp.zeros_like(s2_ref)
        q2_ref[...] = jnp.zeros_like(q2_ref)
    s2_ref[...] += jnp.sum(x2, axis=0, keepdims=True)
    q2_ref[...] += jnp.sum(x2 * x2, axis=0, keepdims=True)


def _mlp3_body(x2_ref, s2_ref, q2_ref, plog_ref, wfh_ref, out_ref):
    m = s2_ref[...] / B
    v = q2_ref[...] / B - m * m
    inv = lax.rsqrt(v + 1e-5)
    h = jnp.maximum((x2_ref[...] - m) * inv, 0.0)
    logit = plog_ref[...] + jnp.sum(h * wfh_ref[...], axis=1, keepdims=True)
    out_ref[...] = jax.nn.sigmoid(logit)


def _const_spec(shape):
    return pl.BlockSpec(shape, lambda i: (0,) * len(shape))


def _batch_spec(cols):
    return pl.BlockSpec((BLK, cols), lambda i: (i, 0))


def _mlp(dense, cat, seq, W1, b1, W2, b2, Wf, bf):
    w1d = W1[:NUM_DENSE]
    w1c = W1[NUM_DENSE:NUM_DENSE + CAT_COLS]
    w1s = W1[NUM_DENSE + CAT_COLS:]
    b1r = b1.reshape(1, HID1)
    b2r = b2.reshape(1, HID2)
    wfw = Wf[: NUM_DENSE + CAT_COLS, 0].reshape(1, -1)
    wfx = Wf[NUM_DENSE + CAT_COLS: NUM_DENSE + CAT_COLS + 4 * EMB, 0].reshape(1, -1)
    wfh = Wf[NUM_DENSE + CAT_COLS + 4 * EMB:, 0].reshape(1, HID2)
    bfr = bf.reshape(1, 1)

    x1, plog, s1, q1 = pl.pallas_call(
        _mlp1_body,
        grid=(NBLK,),
        in_specs=[
            _batch_spec(NUM_DENSE), _batch_spec(CAT_COLS),
            _batch_spec(NUM_SEQ * EMB),
            _const_spec((NUM_DENSE, HID1)), _const_spec((CAT_COLS, HID1)),
            _const_spec((NUM_SEQ * EMB, HID1)),
            _const_spec((1, HID1)), _const_spec((1, NUM_DENSE + CAT_COLS)),
            _const_spec((1, 4 * EMB)), _const_spec((1, 1)),
        ],
        out_specs=[
            _batch_spec(HID1), _batch_spec(1),
            _const_spec((1, HID1)), _const_spec((1, HID1)),
        ],
        out_shape=[
            jax.ShapeDtypeStruct((B, HID1), jnp.float32),
            jax.ShapeDtypeStruct((B, 1), jnp.float32),
            jax.ShapeDtypeStruct((1, HID1), jnp.float32),
            jax.ShapeDtypeStruct((1, HID1), jnp.float32),
        ],
    )(dense, cat, seq, w1d, w1c, w1s, b1r, wfw, wfx, bfr)

    x2, s2, q2 = pl.pallas_call(
        _mlp2_body,
        grid=(NBLK,),
        in_specs=[
            _batch_spec(HID1), _const_spec((1, HID1)), _const_spec((1, HID1)),
            _const_spec((HID1, HID2)), _const_spec((1, HID2)),
        ],
        out_specs=[
            _batch_spec(HID2), _const_spec((1, HID2)), _const_spec((1, HID2)),
        ],
        out_shape=[
            jax.ShapeDtypeStruct((B, HID2), jnp.float32),
            jax.ShapeDtypeStruct((1, HID2), jnp.float32),
            jax.ShapeDtypeStruct((1, HID2), jnp.float32),
        ],
    )(x1, s1, q1, W2, b2r)

    out = pl.pallas_call(
        _mlp3_body,
        grid=(NBLK,),
        in_specs=[
            _batch_spec(HID2), _const_spec((1, HID2)), _const_spec((1, HID2)),
            _batch_spec(1), _const_spec((1, HID2)),
        ],
        out_specs=_batch_spec(1),
        out_shape=jax.ShapeDtypeStruct((B, 1), jnp.float32),
    )(x2, s2, q2, plog, wfh)
    return out


def kernel(dense, cat_idx, seq_idx, cat_tables, seq_tables,
           W1, b1, W2, b2, Wf, bf):
    cat_tabT = jnp.transpose(cat_tables, (0, 2, 1))
    seq_tabT = jnp.transpose(seq_tables, (0, 2, 1))
    app_cat = jnp.concatenate(
        [cat_tables[:, VOCAB - 32:, :].reshape(NUM_CAT, 4, 128),
         jnp.zeros((NUM_CAT, 4, 128), jnp.float32)], axis=1)
    app_seq = jnp.concatenate(
        [seq_tables[:, VOCAB - 32:, :].reshape(NUM_SEQ, 4, 128),
         jnp.zeros((NUM_SEQ, 4, 128), jnp.float32)], axis=1)
    kt_cat, kt_seq = _make_kt()(cat_tabT, seq_tabT, app_cat, app_seq)
    cat_flat = kt_cat.reshape(NUM_CAT * VPAD, EMB)
    seq_tab3 = kt_seq.reshape(NUM_SEQ, VPAD, EMB)

    cat_idx_off = (cat_idx
                   + jnp.arange(NUM_CAT, dtype=jnp.int32) * VPAD).reshape(-1)
    seq_idx_f = seq_idx.reshape(NUM_SEQ, B * SEQ_LEN)

    cat_rows, seq = _make_kg()(cat_idx_off, seq_idx_f, cat_flat, seq_tab3)
    cat = cat_rows.reshape(B, CAT_COLS)
    return _mlp(dense, cat, seq, W1, b1, W2, b2, Wf, bf)


# speedup vs baseline: 1.0810x; 1.0810x over previous
import functools

import jax
import jax.numpy as jnp
from jax import lax
from jax.experimental import pallas as pl
from jax.experimental.pallas import tpu as pltpu
from jax.experimental.pallas import tpu_sc as plsc

B = 16384
NUM_DENSE = 13
NUM_CAT = 26
NUM_SEQ = 2
VOCAB = 100000
EMB = 16
SEQ_LEN = 50
HID1 = 96
HID2 = 32
CAT_COLS = NUM_CAT * EMB

NW = 32
ROWS_W = B // NW

KT_CHUNK = 2048
KT_FULL = 48
KT_TAILV = 1664
RPF = 12504
VPAD = RPF * 8

CAT_SUB = 128
CAT_CHUNK = CAT_SUB * NUM_CAT
SEQ_SUB = 64
SEQ_CHUNK = SEQ_SUB * SEQ_LEN

BLK = 512
NBLK = B // BLK


@functools.cache
def _make_kt():
    mesh = plsc.VectorSubcoreMesh(core_axis_name="c", subcore_axis_name="s")
    return functools.partial(
        pl.kernel,
        out_type=(
            jax.ShapeDtypeStruct((NUM_CAT, RPF, 128), jnp.float32),
            jax.ShapeDtypeStruct((NUM_SEQ, RPF, 128), jnp.float32),
        ),
        mesh=mesh,
        scratch_types=[
            pltpu.VMEM((EMB, KT_CHUNK), jnp.float32),
            pltpu.VMEM((EMB, KT_CHUNK), jnp.float32),
            pltpu.VMEM((KT_CHUNK // 8, 128), jnp.float32),
            pltpu.VMEM((8, 128), jnp.float32),
            pltpu.SemaphoreType.DMA,
            pltpu.SemaphoreType.DMA,
        ],
        compiler_params=pltpu.CompilerParams(
            use_tc_tiling_on_sc=True, needs_layout_passes=False),
    )(_kt_body)


def _kt_field(tab_hbm, app_hbm, out_hbm, f, vin0, vin1, vout, vtail,
              sem0, sem1):
    iota16 = lax.iota(jnp.int32, 16)

    def transpose_chunk(vin, nr):
        def row(r, _):
            base = lax.broadcast_in_dim(r * 8, (16,), ())
            for j in range(8):
                col = plsc.load_gather(vin, [iota16, base + j])
                vout[r, pl.ds(j * EMB, EMB)] = col
            return 0
        lax.fori_loop(0, nr, row, 0)

    def issue(c, vin, sem):
        v0 = c * KT_CHUNK
        pltpu.async_copy(tab_hbm.at[f, pl.ds(0, 8), pl.ds(v0, KT_CHUNK)],
                         vin.at[pl.ds(0, 8), :], sem)
        pltpu.async_copy(tab_hbm.at[f, pl.ds(8, 8), pl.ds(v0, KT_CHUNK)],
                         vin.at[pl.ds(8, 8), :], sem)

    def drain(c, vin, sem):
        v0 = c * KT_CHUNK
        pltpu.make_async_copy(
            tab_hbm.at[f, pl.ds(0, 8), pl.ds(v0, KT_CHUNK)],
            vin.at[pl.ds(0, 8), :], sem).wait()
        pltpu.make_async_copy(
            tab_hbm.at[f, pl.ds(8, 8), pl.ds(v0, KT_CHUNK)],
            vin.at[pl.ds(8, 8), :], sem).wait()

    def flush(c):
        pltpu.sync_copy(vout,
                        out_hbm.at[f, pl.ds(c * (KT_CHUNK // 8),
                                            KT_CHUNK // 8), :])

    issue(0, vin0, sem0)

    def pair(p, _):
        c0 = p * 2
        drain(c0, vin0, sem0)
        issue(c0 + 1, vin1, sem1)
        transpose_chunk(vin0, KT_CHUNK // 8)
        flush(c0)
        drain(c0 + 1, vin1, sem1)

        @pl.when(p < KT_FULL // 2 - 1)
        def _():
            issue(c0 + 2, vin0, sem0)
        transpose_chunk(vin1, KT_CHUNK // 8)
        flush(c0 + 1)
        return 0

    lax.fori_loop(0, KT_FULL // 2, pair, 0)
    v0 = KT_FULL * KT_CHUNK
    pltpu.sync_copy(tab_hbm.at[f, pl.ds(0, 8), pl.ds(v0, KT_TAILV)],
                    vin0.at[pl.ds(0, 8), pl.ds(0, KT_TAILV)])
    pltpu.sync_copy(tab_hbm.at[f, pl.ds(8, 8), pl.ds(v0, KT_TAILV)],
                    vin0.at[pl.ds(8, 8), pl.ds(0, KT_TAILV)])
    transpose_chunk(vin0, KT_TAILV // 8)
    pltpu.sync_copy(vout.at[pl.ds(0, KT_TAILV // 8), :],
                    out_hbm.at[f, pl.ds(v0 // 8, KT_TAILV // 8), :])
    pltpu.sync_copy(app_hbm.at[f], vtail)
    pltpu.sync_copy(vtail, out_hbm.at[f, pl.ds(12496, 8), :])


def _kt_body(cat_tabT_hbm, seq_tabT_hbm, app_cat_hbm, app_seq_hbm,
             cat_out_hbm, seq_out_hbm, vin0, vin1, vout, vtail, sem0, sem1):
    wid = lax.axis_index("s") * 2 + lax.axis_index("c")

    @pl.when(wid < NUM_CAT)
    def _():
        _kt_field(cat_tabT_hbm, app_cat_hbm, cat_out_hbm, wid,
                  vin0, vin1, vout, vtail, sem0, sem1)

    @pl.when(jnp.logical_and(wid >= NUM_CAT, wid < NUM_CAT + NUM_SEQ))
    def _():
        _kt_field(seq_tabT_hbm, app_seq_hbm, seq_out_hbm, wid - NUM_CAT,
                  vin0, vin1, vout, vtail, sem0, sem1)


@functools.cache
def _make_kg():
    mesh = plsc.VectorSubcoreMesh(core_axis_name="c", subcore_axis_name="s")
    return functools.partial(
        pl.kernel,
        out_type=(
            jax.ShapeDtypeStruct((B * NUM_CAT, EMB), jnp.float32),
            jax.ShapeDtypeStruct((B, NUM_SEQ * EMB), jnp.float32),
        ),
        mesh=mesh,
        scratch_types=[
            pltpu.VMEM((CAT_CHUNK,), jnp.int32),
            pltpu.VMEM((CAT_CHUNK, EMB), jnp.float32),
            pltpu.VMEM((SEQ_CHUNK,), jnp.int32),
            pltpu.VMEM((SEQ_CHUNK, EMB), jnp.float32),
            pltpu.VMEM((SEQ_SUB, NUM_SEQ * EMB), jnp.float32),
            pltpu.SemaphoreType.DMA,
        ],
        compiler_params=pltpu.CompilerParams(use_tc_tiling_on_sc=False),
    )(_kg_body)


def _kg_body(cat_idx_hbm, seq_idx_hbm, cat_tab_hbm, seq_tab_hbm,
             cat_out_hbm, seq_out_hbm,
             cidx_v, crows_v, sidx_v, srows_v, pool_v, sem):
    wid = lax.axis_index("s") * 2 + lax.axis_index("c")
    base = wid * ROWS_W

    for k in range(ROWS_W // CAT_SUB):
        off = (base + k * CAT_SUB) * NUM_CAT
        pltpu.sync_copy(cat_idx_hbm.at[pl.ds(off, CAT_CHUNK)], cidx_v)
        pltpu.async_copy(cat_tab_hbm.at[cidx_v], crows_v, sem).wait()
        pltpu.sync_copy(crows_v, cat_out_hbm.at[pl.ds(off, CAT_CHUNK), :])

    for j in range(ROWS_W // SEQ_SUB):
        sub = base + j * SEQ_SUB
        for s in range(NUM_SEQ):
            pltpu.sync_copy(
                seq_idx_hbm.at[s, pl.ds(sub * SEQ_LEN, SEQ_CHUNK)], sidx_v)
            pltpu.async_copy(seq_tab_hbm.at[s].at[sidx_v], srows_v,
                             sem).wait()

            def _pool_row(i, _):
                a0 = srows_v[i * SEQ_LEN, :]
                a1 = srows_v[i * SEQ_LEN + 1, :]
                a2 = srows_v[i * SEQ_LEN + 2, :]
                a3 = srows_v[i * SEQ_LEN + 3, :]
                for l in range(4, SEQ_LEN, 4):
                    a0 = a0 + srows_v[i * SEQ_LEN + l, :]
                    a1 = a1 + srows_v[i * SEQ_LEN + l + 1, :]
                    a2 = a2 + srows_v[i * SEQ_LEN + l + 2, :]
                    a3 = a3 + srows_v[i * SEQ_LEN + l + 3, :]
                pool_v[i, pl.ds(s * EMB, EMB)] = (a0 + a1) + (a2 + a3)
                return 0

            lax.fori_loop(0, SEQ_SUB, _pool_row, 0)
        pltpu.sync_copy(pool_v, seq_out_hbm.at[pl.ds(sub, SEQ_SUB), :])


def _mlp1_body(dense_ref, cat_ref, seq_ref, w1d_ref, w1c_ref, w1s_ref, b1_ref,
               wfw_ref, wfx_ref, bf_ref,
               x1_ref, plog_ref, s1_ref, q1_ref):
    i = pl.program_id(0)
    dense = dense_ref[...]
    cat = cat_ref[...]
    seq = seq_ref[...]
    x1 = (jnp.dot(dense, w1d_ref[...], preferred_element_type=jnp.float32)
          + jnp.dot(cat, w1c_ref[...], preferred_element_type=jnp.float32)
          + jnp.dot(seq, w1s_ref[...], preferred_element_type=jnp.float32)
          + b1_ref[...])
    x1_ref[...] = x1

    cross = jnp.concatenate(
        [cat[:, 0:16] * cat[:, 16:32], cat[:, 32:48] * cat[:, 48:64],
         cat[:, 64:80] * cat[:, 80:96], cat[:, 96:112] * cat[:, 112:128]],
        axis=1)
    plog = (jnp.sum(dense * wfw_ref[:, 0:NUM_DENSE], axis=1, keepdims=True)
            + jnp.sum(cat * wfw_ref[:, NUM_DENSE:], axis=1, keepdims=True)
            + jnp.sum(cross * wfx_ref[...], axis=1, keepdims=True)
            + bf_ref[...])
    plog_ref[...] = plog

    @pl.when(i == 0)
    def _():
        s1_ref[...] = jnp.zeros_like(s1_ref)
        q1_ref[...] = jnp.zeros_like(q1_ref)
    s1_ref[...] += jnp.sum(x1, axis=0, keepdims=True)
    q1_ref[...] += jnp.sum(x1 * x1, axis=0, keepdims=True)


def _mlp2_body(x1_ref, s1_ref, q1_ref, w2_ref, b2_ref,
               x2_ref, s2_ref, q2_ref):
    i = pl.program_id(0)
    m = s1_ref[...] / B
    v = q1_ref[...] / B - m * m
    inv = lax.rsqrt(v + 1e-5)
    h1 = jnp.maximum((x1_ref[...] - m) * inv, 0.0)
    x2 = jnp.dot(h1, w2_ref[...], preferred_element_type=jnp.float32) + b2_ref[...]
    x2_ref[...] = x2

    @pl.when(i == 0)
    def _():
        s2_ref[...] = jnp.zeros_like(s2_ref)
        q2_ref[...] = jnp.zeros_like(q2_ref)
    s2_ref[...] += jnp.sum(x2, axis=0, keepdims=True)
    q2_ref[...] += jnp.sum(x2 * x2, axis=0, keepdims=True)


def _mlp3_body(x2_ref, s2_ref, q2_ref, plog_ref, wfh_ref, out_ref):
    m = s2_ref[...] / B
    v = q2_ref[...] / B - m * m
    inv = lax.rsqrt(v + 1e-5)
    h = jnp.maximum((x2_ref[...] - m) * inv, 0.0)
    logit = plog_ref[...] + jnp.sum(h * wfh_ref[...], axis=1, keepdims=True)
    out_ref[...] = jax.nn.sigmoid(logit)


def _const_spec(shape):
    return pl.BlockSpec(shape, lambda i: (0,) * len(shape))


def _batch_spec(cols):
    return pl.BlockSpec((BLK, cols), lambda i: (i, 0))


def _mlp(dense, cat, seq, W1, b1, W2, b2, Wf, bf):
    w1d = W1[:NUM_DENSE]
    w1c = W1[NUM_DENSE:NUM_DENSE + CAT_COLS]
    w1s = W1[NUM_DENSE + CAT_COLS:]
    b1r = b1.reshape(1, HID1)
    b2r = b2.reshape(1, HID2)
    wfw = Wf[: NUM_DENSE + CAT_COLS, 0].reshape(1, -1)
    wfx = Wf[NUM_DENSE + CAT_COLS: NUM_DENSE + CAT_COLS + 4 * EMB, 0].reshape(1, -1)
    wfh = Wf[NUM_DENSE + CAT_COLS + 4 * EMB:, 0].reshape(1, HID2)
    bfr = bf.reshape(1, 1)

    x1, plog, s1, q1 = pl.pallas_call(
        _mlp1_body,
        grid=(NBLK,),
        in_specs=[
            _batch_spec(NUM_DENSE), _batch_spec(CAT_COLS),
            _batch_spec(NUM_SEQ * EMB),
            _const_spec((NUM_DENSE, HID1)), _const_spec((CAT_COLS, HID1)),
            _const_spec((NUM_SEQ * EMB, HID1)),
            _const_spec((1, HID1)), _const_spec((1, NUM_DENSE + CAT_COLS)),
            _const_spec((1, 4 * EMB)), _const_spec((1, 1)),
        ],
        out_specs=[
            _batch_spec(HID1), _batch_spec(1),
            _const_spec((1, HID1)), _const_spec((1, HID1)),
        ],
        out_shape=[
            jax.ShapeDtypeStruct((B, HID1), jnp.float32),
            jax.ShapeDtypeStruct((B, 1), jnp.float32),
            jax.ShapeDtypeStruct((1, HID1), jnp.float32),
            jax.ShapeDtypeStruct((1, HID1), jnp.float32),
        ],
    )(dense, cat, seq, w1d, w1c, w1s, b1r, wfw, wfx, bfr)

    x2, s2, q2 = pl.pallas_call(
        _mlp2_body,
        grid=(NBLK,),
        in_specs=[
            _batch_spec(HID1), _const_spec((1, HID1)), _const_spec((1, HID1)),
            _const_spec((HID1, HID2)), _const_spec((1, HID2)),
        ],
        out_specs=[
            _batch_spec(HID2), _const_spec((1, HID2)), _const_spec((1, HID2)),
        ],
        out_shape=[
            jax.ShapeDtypeStruct((B, HID2), jnp.float32),
            jax.ShapeDtypeStruct((1, HID2), jnp.float32),
            jax.ShapeDtypeStruct((1, HID2), jnp.float32),
        ],
    )(x1, s1, q1, W2, b2r)

    out = pl.pallas_call(
        _mlp3_body,
        grid=(NBLK,),
        in_specs=[
            _batch_spec(HID2), _const_spec((1, HID2)), _const_spec((1, HID2)),
            _batch_spec(1), _const_spec((1, HID2)),
        ],
        out_specs=_batch_spec(1),
        out_shape=jax.ShapeDtypeStruct((B, 1), jnp.float32),
    )(x2, s2, q2, plog, wfh)
    return out


def kernel(dense, cat_idx, seq_idx, cat_tables, seq_tables,
           W1, b1, W2, b2, Wf, bf):
    cat_tabT = jnp.transpose(cat_tables, (0, 2, 1))
    seq_tabT = jnp.transpose(seq_tables, (0, 2, 1))
    app_cat = jnp.concatenate(
        [cat_tables[:, VOCAB - 32:, :].reshape(NUM_CAT, 4, 128),
         jnp.zeros((NUM_CAT, 4, 128), jnp.float32)], axis=1)
    app_seq = jnp.concatenate(
        [seq_tables[:, VOCAB - 32:, :].reshape(NUM_SEQ, 4, 128),
         jnp.zeros((NUM_SEQ, 4, 128), jnp.float32)], axis=1)
    kt_cat, kt_seq = _make_kt()(cat_tabT, seq_tabT, app_cat, app_seq)
    cat_flat = kt_cat.reshape(NUM_CAT * VPAD, EMB)
    seq_tab3 = kt_seq.reshape(NUM_SEQ, VPAD, EMB)

    cat_idx_off = (cat_idx
                   + jnp.arange(NUM_CAT, dtype=jnp.int32) * VPAD).reshape(-1)
    seq_idx_f = seq_idx.reshape(NUM_SEQ, B * SEQ_LEN)

    cat_rows, seq = _make_kg()(cat_idx_off, seq_idx_f, cat_flat, seq_tab3)
    cat = cat_rows.reshape(B, CAT_COLS)
    return _mlp(dense, cat, seq, W1, b1, W2, b2, Wf, bf)
